# baseline (speedup 1.0000x reference)
import functools

import jax
import jax.numpy as jnp
from jax import lax
from jax.experimental import pallas as pl
from jax.experimental.pallas import tpu as pltpu
from jax.experimental.pallas import tpu_sc as plsc

TEMP = 0.07
B, N, C, K = 8, 196, 384, 128
SLAB = 56
SLAB_LAST = 32
SLABS_PER_B = 4
NSTRIDE = 200
OUT_LEN = B * NSTRIDE


def _sim_body(qt_ref, kt_ref, negt_ref, pos_ref, s_ref, negp_ref, posp_ref):
    for b in range(B):
        qb = qt_ref[:, b, :]
        kb = kt_ref[:, b, :]
        qn = qb / jnp.maximum(jnp.sqrt(jnp.sum(qb * qb, axis=-1, keepdims=True)),
                              1e-12)
        kn = kb / jnp.maximum(jnp.sqrt(jnp.sum(kb * kb, axis=-1, keepdims=True)),
                              1e-12)
        s = lax.dot_general(qn.astype(jnp.bfloat16), kn.astype(jnp.bfloat16),
                            (((1,), (1,)), ((), ())),
                            preferred_element_type=jnp.float32)
        s_ref[b, 0:N, :] = s / TEMP
        negp_ref[b, 0:N, :] = negt_ref[:, b, :]
        posp_ref[pl.ds(b * NSTRIDE, N)] = pos_ref[b, :]


def _loss_body(x_ref, pv_ref, o_ref):
    x = x_ref[:]
    pv = pv_ref[:].reshape(OUT_LEN, 1)
    i = lax.broadcasted_iota(jnp.int32, (OUT_LEN, 1), 0)
    valid = (i % NSTRIDE) < N
    m = jnp.maximum(jnp.max(x, axis=1, keepdims=True), pv)
    se = jnp.sum(jnp.exp(x - m), axis=1, keepdims=True) + jnp.exp(pv - m)
    per_row = jnp.where(valid, m + jnp.log(se) - pv, 0.0)
    o_ref[:, :] = (jnp.sum(per_row) / (B * N)).reshape(1, 1)


def _slab_work(nrows, nreal, b, p0, s_hbm, pos_hbm, neg_hbm, out_hbm, pv_hbm,
               s_v, pos_v, neg_v, out_v, pv_v):
    pltpu.sync_copy(s_hbm.at[b, pl.ds(p0, nrows)], s_v.at[pl.ds(0, nrows)])
    pltpu.sync_copy(neg_hbm.at[b, pl.ds(p0, nrows)], neg_v.at[pl.ds(0, nrows)])
    pltpu.sync_copy(pos_hbm.at[pl.ds(b * NSTRIDE + p0, nrows)],
                    pos_v.at[pl.ds(0, nrows)])

    @plsc.parallel_loop(0, nreal, unroll=8)
    def _(r):
        ridx = jnp.full((16,), r, jnp.int32)
        for g in range(K // 16):
            cols = neg_v[r, pl.ds(g * 16, 16)]
            out_v[pl.ds(r * K + g * 16, 16)] = plsc.load_gather(s_v, [ridx, cols])

    lanes = lax.iota(jnp.int32, 16)
    for t in range((nreal + 15) // 16):
        rows = lanes + t * 16
        ok = rows < nreal
        rows_c = jnp.where(ok, rows, 0)
        pc = jnp.where(ok, pos_v[pl.ds(t * 16, 16)], 0)
        pv_v[pl.ds(t * 16, 16)] = plsc.load_gather(s_v, [rows_c, pc])

    base = b * NSTRIDE + p0
    pltpu.sync_copy(out_v.at[pl.ds(0, nrows * K)],
                    out_hbm.at[pl.ds(base * K, nrows * K)])
    pltpu.sync_copy(pv_v.at[pl.ds(0, nrows)], pv_hbm.at[pl.ds(base, nrows)])


def _gather_body(s_hbm, pos_hbm, neg_hbm, out_hbm, pv_hbm,
                 s_v, pos_v, neg_v, out_v, pv_v):
    nc = plsc.get_sparse_core_info().num_cores
    wid = lax.axis_index("s") * nc + lax.axis_index("c")
    b = wid // SLABS_PER_B
    slab = wid % SLABS_PER_B
    p0 = slab * SLAB
    refs = (s_hbm, pos_hbm, neg_hbm, out_hbm, pv_hbm,
            s_v, pos_v, neg_v, out_v, pv_v)

    @pl.when(slab < SLABS_PER_B - 1)
    def _():
        _slab_work(SLAB, SLAB, b, p0, *refs)

    @pl.when(slab == SLABS_PER_B - 1)
    def _():
        _slab_work(SLAB_LAST, N - (SLABS_PER_B - 1) * SLAB, b, p0, *refs)


@functools.cache
def _gather_call():
    return pl.kernel(
        _gather_body,
        mesh=plsc.VectorSubcoreMesh(core_axis_name="c", subcore_axis_name="s"),
        out_type=(jax.ShapeDtypeStruct((OUT_LEN * K,), jnp.float32),
                  jax.ShapeDtypeStruct((OUT_LEN,), jnp.float32)),
        scratch_types=[
            pltpu.VMEM((SLAB, N), jnp.float32),
            pltpu.VMEM((64,), jnp.int32),
            pltpu.VMEM((SLAB, K), jnp.int32),
            pltpu.VMEM((SLAB * K,), jnp.float32),
            pltpu.VMEM((64,), jnp.float32),
        ],
        compiler_params=pltpu.CompilerParams(needs_layout_passes=False),
    )


def kernel(q, k, positive_indices, negative_indices):
    s, neg, pos = pl.pallas_call(
        _sim_body,
        out_shape=(jax.ShapeDtypeStruct((B, NSTRIDE, N), jnp.float32),
                   jax.ShapeDtypeStruct((B, NSTRIDE, K), jnp.int32),
                   jax.ShapeDtypeStruct((OUT_LEN,), jnp.int32)),
    )(jnp.transpose(q, (1, 0, 2)), jnp.transpose(k, (1, 0, 2)),
      jnp.transpose(negative_indices.astype(jnp.int32), (1, 0, 2)),
      positive_indices.astype(jnp.int32))

    negs, pv = _gather_call()(s, pos, neg)

    loss = pl.pallas_call(
        _loss_body,
        out_shape=jax.ShapeDtypeStruct((1, 1), jnp.float32),
    )(negs.reshape(OUT_LEN, K), pv)
    return loss[0, 0]


# speedup vs baseline: 1.1504x; 1.1504x over previous
import functools

import jax
import jax.numpy as jnp
from jax import lax
from jax.experimental import pallas as pl
from jax.experimental.pallas import tpu as pltpu
from jax.experimental.pallas import tpu_sc as plsc

TEMP = 0.07
B, N, C, K = 8, 196, 384, 128
SLAB = 56
SLAB_LAST = 32
SLABS_PER_B = 4
NSTRIDE = 200
OUT_LEN = B * NSTRIDE


def _sim_body(qt_ref, kt_ref, negt_ref, pos_ref, s_ref, negp_ref, posp_ref):
    for b in range(B):
        qb = qt_ref[:, b, :]
        kb = kt_ref[:, b, :]
        qn = qb / jnp.maximum(jnp.sqrt(jnp.sum(qb * qb, axis=-1, keepdims=True)),
                              1e-12)
        kn = kb / jnp.maximum(jnp.sqrt(jnp.sum(kb * kb, axis=-1, keepdims=True)),
                              1e-12)
        s = lax.dot_general(qn, kn, (((1,), (1,)), ((), ())),
                            preferred_element_type=jnp.float32)
        s_ref[b, 0:N, :] = s / TEMP
        negp_ref[b, 0:N, :] = negt_ref[:, b, :]
        posp_ref[pl.ds(b * NSTRIDE, N)] = pos_ref[b, :]


def _loss_body(x_ref, pv_ref, o_ref):
    x = x_ref[:]
    pv = pv_ref[:].reshape(OUT_LEN, 1)
    i = lax.broadcasted_iota(jnp.int32, (OUT_LEN, 1), 0)
    valid = (i % NSTRIDE) < N
    m = jnp.maximum(jnp.max(x, axis=1, keepdims=True), pv)
    se = jnp.sum(jnp.exp(x - m), axis=1, keepdims=True) + jnp.exp(pv - m)
    per_row = jnp.where(valid, m + jnp.log(se) - pv, 0.0)
    o_ref[:, :] = (jnp.sum(per_row) / (B * N)).reshape(1, 1)


def _slab_work(nrows, nreal, b, p0, s_hbm, pos_hbm, neg_hbm, out_hbm, pv_hbm,
               s_v, pos_v, neg_v, out_v, pv_v):
    pltpu.sync_copy(s_hbm.at[b, pl.ds(p0, nrows)], s_v.at[pl.ds(0, nrows)])
    pltpu.sync_copy(neg_hbm.at[b, pl.ds(p0, nrows)], neg_v.at[pl.ds(0, nrows)])
    pltpu.sync_copy(pos_hbm.at[pl.ds(b * NSTRIDE + p0, nrows)],
                    pos_v.at[pl.ds(0, nrows)])

    @plsc.parallel_loop(0, nreal, unroll=8)
    def _(r):
        ridx = jnp.full((16,), r, jnp.int32)
        for g in range(K // 16):
            cols = neg_v[r, pl.ds(g * 16, 16)]
            out_v[pl.ds(r * K + g * 16, 16)] = plsc.load_gather(s_v, [ridx, cols])

    lanes = lax.iota(jnp.int32, 16)
    for t in range((nreal + 15) // 16):
        rows = lanes + t * 16
        ok = rows < nreal
        rows_c = jnp.where(ok, rows, 0)
        pc = jnp.where(ok, pos_v[pl.ds(t * 16, 16)], 0)
        pv_v[pl.ds(t * 16, 16)] = plsc.load_gather(s_v, [rows_c, pc])

    base = b * NSTRIDE + p0
    pltpu.sync_copy(out_v.at[pl.ds(0, nrows * K)],
                    out_hbm.at[pl.ds(base * K, nrows * K)])
    pltpu.sync_copy(pv_v.at[pl.ds(0, nrows)], pv_hbm.at[pl.ds(base, nrows)])


def _gather_body(s_hbm, pos_hbm, neg_hbm, out_hbm, pv_hbm,
                 s_v, pos_v, neg_v, out_v, pv_v):
    nc = plsc.get_sparse_core_info().num_cores
    wid = lax.axis_index("s") * nc + lax.axis_index("c")
    b = wid // SLABS_PER_B
    slab = wid % SLABS_PER_B
    p0 = slab * SLAB
    refs = (s_hbm, pos_hbm, neg_hbm, out_hbm, pv_hbm,
            s_v, pos_v, neg_v, out_v, pv_v)

    @pl.when(slab < SLABS_PER_B - 1)
    def _():
        _slab_work(SLAB, SLAB, b, p0, *refs)

    @pl.when(slab == SLABS_PER_B - 1)
    def _():
        _slab_work(SLAB_LAST, N - (SLABS_PER_B - 1) * SLAB, b, p0, *refs)


@functools.cache
def _gather_call():
    return pl.kernel(
        _gather_body,
        mesh=plsc.VectorSubcoreMesh(core_axis_name="c", subcore_axis_name="s"),
        out_type=(jax.ShapeDtypeStruct((OUT_LEN * K,), jnp.float32),
                  jax.ShapeDtypeStruct((OUT_LEN,), jnp.float32)),
        scratch_types=[
            pltpu.VMEM((SLAB, N), jnp.float32),
            pltpu.VMEM((64,), jnp.int32),
            pltpu.VMEM((SLAB, K), jnp.int32),
            pltpu.VMEM((SLAB * K,), jnp.float32),
            pltpu.VMEM((64,), jnp.float32),
        ],
        compiler_params=pltpu.CompilerParams(needs_layout_passes=False),
    )


def kernel(q, k, positive_indices, negative_indices):
    s, neg, pos = pl.pallas_call(
        _sim_body,
        out_shape=(jax.ShapeDtypeStruct((B, NSTRIDE, N), jnp.float32),
                   jax.ShapeDtypeStruct((B, NSTRIDE, K), jnp.int32),
                   jax.ShapeDtypeStruct((OUT_LEN,), jnp.int32)),
    )(jnp.transpose(q, (1, 0, 2)), jnp.transpose(k, (1, 0, 2)),
      jnp.transpose(negative_indices.astype(jnp.int32), (1, 0, 2)),
      positive_indices.astype(jnp.int32))

    negs, pv = _gather_call()(s, pos, neg)

    loss = pl.pallas_call(
        _loss_body,
        out_shape=jax.ShapeDtypeStruct((1, 1), jnp.float32),
    )(negs.reshape(OUT_LEN, K), pv)
    return loss[0, 0]


# speedup vs baseline: 1.1958x; 1.0395x over previous
import functools

import jax
import jax.numpy as jnp
from jax import lax
from jax.experimental import pallas as pl
from jax.experimental.pallas import tpu as pltpu
from jax.experimental.pallas import tpu_sc as plsc

TEMP = 0.07
B, N, C, K = 8, 196, 384, 128
SLAB = 56
SLABS_PER_B = 4
NSTRIDE = SLAB * SLABS_PER_B
OUT_LEN = B * NSTRIDE


def _sim_body(qt_ref, kt_ref, negt_ref, pos_ref, s_ref, negp_ref, posp_ref):
    for b in range(B):
        qb = qt_ref[:, b, :]
        kb = kt_ref[:, b, :]
        qn = qb / jnp.maximum(jnp.sqrt(jnp.sum(qb * qb, axis=-1, keepdims=True)),
                              1e-12)
        kn = kb / jnp.maximum(jnp.sqrt(jnp.sum(kb * kb, axis=-1, keepdims=True)),
                              1e-12)
        s = lax.dot_general(qn, kn, (((1,), (1,)), ((), ())),
                            preferred_element_type=jnp.float32)
        s_ref[b, 0:N, :] = s / TEMP
        negp_ref[b, 192:NSTRIDE, :] = jnp.zeros((NSTRIDE - 192, K), jnp.int32)
        negp_ref[b, 0:N, :] = negt_ref[:, b, :]
        posp_ref[pl.ds(b * NSTRIDE + 192, NSTRIDE - 192)] = jnp.zeros(
            (NSTRIDE - 192,), jnp.int32)
        posp_ref[pl.ds(b * NSTRIDE, N)] = pos_ref[b, :]


def _loss_body(x_ref, pv_ref, o_ref):
    x = x_ref[:]
    pv = pv_ref[:].reshape(OUT_LEN, 1)
    i = lax.broadcasted_iota(jnp.int32, (OUT_LEN, 1), 0)
    valid = (i % NSTRIDE) < N
    m = jnp.maximum(jnp.max(x, axis=1, keepdims=True), pv)
    se = jnp.sum(jnp.exp(x - m), axis=1, keepdims=True) + jnp.exp(pv - m)
    per_row = jnp.where(valid, m + jnp.log(se) - pv, 0.0)
    o_ref[:, :] = (jnp.sum(per_row) / (B * N)).reshape(1, 1)


def _gather_body(s_hbm, pos_hbm, neg_hbm, out_hbm, pv_hbm,
                 s_v, pos_v, neg_v, out_v, pv_v, sem):
    nc = plsc.get_sparse_core_info().num_cores
    wid = lax.axis_index("s") * nc + lax.axis_index("c")
    b = wid // SLABS_PER_B
    p0 = (wid % SLABS_PER_B) * SLAB

    c1 = pltpu.make_async_copy(s_hbm.at[b, pl.ds(p0, SLAB)], s_v, sem)
    c2 = pltpu.make_async_copy(neg_hbm.at[b, pl.ds(p0, SLAB)], neg_v, sem)
    c3 = pltpu.make_async_copy(pos_hbm.at[pl.ds(b * NSTRIDE + p0, SLAB)],
                               pos_v.at[pl.ds(0, SLAB)], sem)
    c1.start()
    c2.start()
    c3.start()
    c1.wait()
    c2.wait()
    c3.wait()

    @plsc.parallel_loop(0, SLAB, unroll=4)
    def _(r):
        ridx = jnp.full((16,), r, jnp.int32)
        for g in range(K // 16):
            cols = neg_v[r, pl.ds(g * 16, 16)]
            out_v[pl.ds(r * K + g * 16, 16)] = plsc.load_gather(s_v, [ridx, cols])

    lanes = lax.iota(jnp.int32, 16)
    for t in range((SLAB + 15) // 16):
        rows = lanes + t * 16
        ok = rows < SLAB
        rows_c = jnp.where(ok, rows, 0)
        pc = jnp.where(ok, pos_v[pl.ds(t * 16, 16)], 0)
        pv_v[pl.ds(t * 16, 16)] = plsc.load_gather(s_v, [rows_c, pc])

    base = b * NSTRIDE + p0
    c4 = pltpu.make_async_copy(out_v, out_hbm.at[pl.ds(base * K, SLAB * K)], sem)
    c5 = pltpu.make_async_copy(pv_v.at[pl.ds(0, SLAB)],
                               pv_hbm.at[pl.ds(base, SLAB)], sem)
    c4.start()
    c5.start()
    c4.wait()
    c5.wait()


@functools.cache
def _gather_call():
    return pl.kernel(
        _gather_body,
        mesh=plsc.VectorSubcoreMesh(core_axis_name="c", subcore_axis_name="s"),
        out_type=(jax.ShapeDtypeStruct((OUT_LEN * K,), jnp.float32),
                  jax.ShapeDtypeStruct((OUT_LEN,), jnp.float32)),
        scratch_types=[
            pltpu.VMEM((SLAB, N), jnp.float32),
            pltpu.VMEM((64,), jnp.int32),
            pltpu.VMEM((SLAB, K), jnp.int32),
            pltpu.VMEM((SLAB * K,), jnp.float32),
            pltpu.VMEM((64,), jnp.float32),
            pltpu.SemaphoreType.DMA,
        ],
        compiler_params=pltpu.CompilerParams(needs_layout_passes=False),
    )


def kernel(q, k, positive_indices, negative_indices):
    s, neg, pos = pl.pallas_call(
        _sim_body,
        out_shape=(jax.ShapeDtypeStruct((B, NSTRIDE, N), jnp.float32),
                   jax.ShapeDtypeStruct((B, NSTRIDE, K), jnp.int32),
                   jax.ShapeDtypeStruct((OUT_LEN,), jnp.int32)),
    )(jnp.transpose(q, (1, 0, 2)), jnp.transpose(k, (1, 0, 2)),
      jnp.transpose(negative_indices.astype(jnp.int32), (1, 0, 2)),
      positive_indices.astype(jnp.int32))

    negs, pv = _gather_call()(s, pos, neg)

    loss = pl.pallas_call(
        _loss_body,
        out_shape=jax.ShapeDtypeStruct((1, 1), jnp.float32),
    )(negs.reshape(OUT_LEN, K), pv)
    return loss[0, 0]
